# baseline (speedup 1.0000x reference)
import functools

import jax
import jax.numpy as jnp
from jax import lax
from jax.experimental import pallas as pl
from jax.experimental.pallas import tpu as pltpu
from jax.experimental.pallas import tpu_sc as plsc

N = 320000
D = 128
K = 40
S = 10000
HP = 48

NCHUNK = 2
NR = N // NCHUNK

NC = 2
NS = 16
NW = NC * NS
GRP = 128
NGRP = NR // GRP
SEG_W = 320
SPG = NW * SEG_W
ACC_R = SEG_W + 8
SEARCH_IT = max(NGRP - 1, 1).bit_length()

ROW_BLK = 3200


def _mlp1_body(x_ref, w_ref, b_ref, o_ref):
    y = jnp.dot(x_ref[...], w_ref[...], preferred_element_type=jnp.float32)
    o_ref[...] = jnp.maximum(y + b_ref[...], 0.0)


@functools.cache
def _make_mlp1(chunk):
    nblk = NR // ROW_BLK
    return pl.pallas_call(
        _mlp1_body,
        grid=(nblk,),
        in_specs=[
            pl.BlockSpec((ROW_BLK, D), lambda i, c=chunk: (i + c * nblk, 0)),
            pl.BlockSpec((D, HP), lambda i: (0, 0)),
            pl.BlockSpec((1, HP), lambda i: (0, 0)),
        ],
        out_specs=pl.BlockSpec((ROW_BLK, HP), lambda i: (i, 0)),
        out_shape=jax.ShapeDtypeStruct((NR, HP), jnp.float32),
        compiler_params=pltpu.CompilerParams(dimension_semantics=("arbitrary",)),
    )


@functools.cache
def _make_segsum(chunk):
    row_off = chunk * NR
    mesh = plsc.VectorSubcoreMesh(
        core_axis_name="c", subcore_axis_name="s", num_cores=NC, num_subcores=NS
    )
    return pl.kernel(
        functools.partial(_segsum_body, row_off),
        out_type=jax.ShapeDtypeStruct((SPG, HP), jnp.float32),
        mesh=mesh,
        compiler_params=pltpu.CompilerParams(use_tc_tiling_on_sc=True),
        scratch_types=[
            pltpu.VMEM((2, 16), jnp.int32),
            pltpu.SemaphoreType.DMA,
            pltpu.SemaphoreType.DMA,
            pltpu.VMEM((2, GRP), jnp.int32),
            pltpu.VMEM((2, GRP), jnp.int32),
            pltpu.VMEM((2, GRP, HP), jnp.float32),
            pltpu.SemaphoreType.DMA,
            pltpu.SemaphoreType.DMA,
            pltpu.VMEM((ACC_R, HP), jnp.float32),
            pltpu.VMEM_SHARED((NS * ACC_R, HP), jnp.float32),
        ],
    )


def _segsum_body(row_off, h_hbm, seg_hbm, out_hbm, probe2, psem0, psem1, seg2,
                 idx2, rows2, gsem0, gsem1, zbuf_v, acc):
    cid = lax.axis_index("c")
    sid = lax.axis_index("s")
    wid = cid * NS + sid
    base = wid * SEG_W
    accbase = sid * ACC_R
    gsems = (gsem0, gsem1)

    zero16 = jnp.zeros((16,), jnp.float32)

    def zrow(i, carry):
        for j in range(HP // 16):
            zbuf_v[i, pl.ds(j * 16, 16)] = zero16
        return carry

    lax.fori_loop(0, ACC_R, zrow, 0)
    pltpu.sync_copy(zbuf_v, acc.at[pl.ds(accbase, ACC_R)])

    def sit(_, st):
        lo0, hi0, lo1, hi1 = st
        a0 = lo0 < hi0
        a1 = lo1 < hi1
        m0 = jnp.minimum((lo0 + hi0) // 2, NGRP - 1)
        m1 = jnp.minimum((lo1 + hi1) // 2, NGRP - 1)
        c0 = pltpu.async_copy(
            seg_hbm.at[pl.ds(row_off + m0 * GRP + GRP - 16, 16)],
            probe2.at[0], psem0,
        )
        c1 = pltpu.async_copy(
            seg_hbm.at[pl.ds(row_off + m1 * GRP, 16)], probe2.at[1], psem1
        )
        c0.wait()
        c1.wait()
        p0 = probe2[0, pl.ds(0, 16)][15] >= base
        p1 = probe2[1, pl.ds(0, 16)][0] >= base + SEG_W
        lo0n = jnp.where(a0 & jnp.logical_not(p0), m0 + 1, lo0)
        hi0n = jnp.where(a0 & p0, m0, hi0)
        lo1n = jnp.where(a1 & jnp.logical_not(p1), m1 + 1, lo1)
        hi1n = jnp.where(a1 & p1, m1, hi1)
        return (lo0n, hi0n, lo1n, hi1n)

    g_lo, _, g_hi, _ = lax.fori_loop(
        0, SEARCH_IT, sit,
        (jnp.int32(0), jnp.int32(NGRP), jnp.int32(0), jnp.int32(NGRP)),
    )

    def issue(g, b):
        pltpu.async_copy(
            seg_hbm.at[pl.ds(row_off + g * GRP, GRP)], seg2.at[b], gsems[b]
        )
        pltpu.async_copy(h_hbm.at[pl.ds(g * GRP, GRP)], rows2.at[b], gsems[b])

    def wait(b):
        pltpu.make_async_copy(
            seg_hbm.at[pl.ds(0, GRP)], seg2.at[b], gsems[b]
        ).wait()
        pltpu.make_async_copy(
            h_hbm.at[pl.ds(0, GRP)], rows2.at[b], gsems[b]
        ).wait()

    @pl.when(g_lo < g_hi)
    def _():
        issue(g_lo, 0)

    def gbody(i, carry):
        g0 = g_lo + 2 * i
        for b in range(2):
            g = g0 + b

            @pl.when(g < g_hi)
            def _():
                @pl.when(g + 1 < g_hi)
                def _():
                    issue(g + 1, 1 - b)

                wait(b)
                for j in range(GRP // 16):
                    sv = seg2[b, pl.ds(j * 16, 16)]
                    loc = sv - base
                    ok = (loc >= 0) & (loc < SEG_W)
                    idx2[b, pl.ds(j * 16, 16)] = (
                        jnp.where(ok, loc, SEG_W) + accbase
                    )
                pltpu.sync_copy(rows2.at[b], acc.at[idx2.at[b]], add=True)

        return carry

    lax.fori_loop(0, (g_hi - g_lo + 1) // 2, gbody, 0)

    pltpu.sync_copy(
        acc.at[pl.ds(accbase, SEG_W)],
        out_hbm.at[pl.ds(base, SEG_W)],
    )


def _head_body(p0_ref, p1_ref, w2_ref, b2_ref, w3_ref, b3_ref, w4_ref, b4_ref,
               logits_ref, probs_ref):
    s = p0_ref[...] + p1_ref[...]
    sums = s[:S, :K]
    cnt = s[:S, K:K + 1]
    mean = sums / jnp.maximum(cnt, 1.0)
    x = jnp.dot(mean, w2_ref[...], preferred_element_type=jnp.float32) + b2_ref[...]
    x = jnp.dot(x, w3_ref[...], preferred_element_type=jnp.float32) + b3_ref[...]
    x = jnp.maximum(x, 0.0)
    logits = jnp.dot(x, w4_ref[...], preferred_element_type=jnp.float32) + b4_ref[...]
    m = jnp.max(logits, axis=-1, keepdims=True)
    e = jnp.exp(logits - m)
    probs = e / jnp.sum(e, axis=-1, keepdims=True)
    logits_ref[...] = logits
    probs_ref[...] = probs


_head = pl.pallas_call(
    _head_body,
    out_shape=(
        jax.ShapeDtypeStruct((S, 2), jnp.float32),
        jax.ShapeDtypeStruct((S, 2), jnp.float32),
    ),
)


def kernel(features, segments, W1, b1, W2, b2, W3, b3, W4, b4):
    W1p = jnp.concatenate([W1, jnp.zeros((D, HP - K), jnp.float32)], axis=1)
    b1p = jnp.concatenate(
        [b1, jnp.ones((1,), jnp.float32), jnp.zeros((HP - K - 1,), jnp.float32)]
    )[None, :]
    partials = []
    for c in range(NCHUNK):
        h_c = _make_mlp1(c)(features, W1p, b1p)
        partials.append(_make_segsum(c)(h_c, segments))
    logits, probs = _head(
        partials[0], partials[1],
        W2, b2[None, :], W3, b3[None, :], W4, b4[None, :],
    )
    return (logits, probs)


# speedup vs baseline: 1.2884x; 1.2884x over previous
import functools

import jax
import jax.numpy as jnp
from jax import lax
from jax.experimental import pallas as pl
from jax.experimental.pallas import tpu as pltpu
from jax.experimental.pallas import tpu_sc as plsc

N = 320000
D = 128
K = 40
S = 10000
HP = 48

NCHUNK = 2
NR = N // NCHUNK

NC = 2
NS = 16
NW = NC * NS
GRP = 128
NGRP = NR // GRP
SEG_W = 320
HW = SEG_W // 2
SPG = NW * SEG_W
HALF = SPG // 2
ACC_R = SEG_W + 8
SEARCH_IT = max(NGRP - 1, 1).bit_length()

ROW_BLK = 3200


def _mlp1_body(x_ref, w_ref, b_ref, o_ref):
    y = jnp.dot(x_ref[...], w_ref[...], preferred_element_type=jnp.float32)
    o_ref[...] = jnp.maximum(y + b_ref[...], 0.0)


@functools.cache
def _make_mlp1(chunk):
    nblk = NR // ROW_BLK
    return pl.pallas_call(
        _mlp1_body,
        grid=(nblk,),
        in_specs=[
            pl.BlockSpec((ROW_BLK, D), lambda i, c=chunk: (i + c * nblk, 0)),
            pl.BlockSpec((D, HP), lambda i: (0, 0)),
            pl.BlockSpec((1, HP), lambda i: (0, 0)),
        ],
        out_specs=pl.BlockSpec((ROW_BLK, HP), lambda i: (i, 0)),
        out_shape=jax.ShapeDtypeStruct((NR, HP), jnp.float32),
        compiler_params=pltpu.CompilerParams(dimension_semantics=("arbitrary",)),
    )


@functools.cache
def _make_segsum(chunk):
    row_off = chunk * NR
    mesh = plsc.VectorSubcoreMesh(
        core_axis_name="c", subcore_axis_name="s", num_cores=NC, num_subcores=NS
    )
    return pl.kernel(
        functools.partial(_segsum_body, row_off),
        out_type=jax.ShapeDtypeStruct((SPG, HP), jnp.float32),
        mesh=mesh,
        compiler_params=pltpu.CompilerParams(use_tc_tiling_on_sc=True),
        scratch_types=[
            pltpu.VMEM((4, 16), jnp.int32),
            pltpu.SemaphoreType.DMA,
            pltpu.SemaphoreType.DMA,
            pltpu.SemaphoreType.DMA,
            pltpu.SemaphoreType.DMA,
            pltpu.VMEM((2, GRP), jnp.int32),
            pltpu.VMEM((2, GRP), jnp.int32),
            pltpu.VMEM((2, GRP, HP), jnp.float32),
            pltpu.SemaphoreType.DMA,
            pltpu.SemaphoreType.DMA,
            pltpu.VMEM((ACC_R, HP), jnp.float32),
            pltpu.VMEM_SHARED((NS * ACC_R, HP), jnp.float32),
        ],
    )


def _segsum_body(row_off, h_hbm, seg_hbm, out_hbm, probe4, psem0, psem1,
                 psem2, psem3, seg2, idx2, rows2, gsem0, gsem1, zbuf_v, acc):
    cid = lax.axis_index("c")
    sid = lax.axis_index("s")
    wid = cid * NS + sid
    base1 = wid * HW
    base2 = HALF + wid * HW
    accbase = sid * ACC_R
    gsems = (gsem0, gsem1)

    zero16 = jnp.zeros((16,), jnp.float32)

    def zrow(i, carry):
        for j in range(HP // 16):
            zbuf_v[i, pl.ds(j * 16, 16)] = zero16
        return carry

    lax.fori_loop(0, ACC_R, zrow, 0)
    pltpu.sync_copy(zbuf_v, acc.at[pl.ds(accbase, ACC_R)])

    psems = (psem0, psem1, psem2, psem3)
    offs = (GRP - 16, 0, GRP - 16, 0)
    lanes = (15, 0, 15, 0)
    thresholds = (base1, base1 + HW, base2, base2 + HW)

    def sit(_, st):
        los = st[0::2]
        his = st[1::2]
        mids = []
        copies = []
        for k in range(4):
            mids.append(jnp.minimum((los[k] + his[k]) // 2, NGRP - 1))
            copies.append(pltpu.async_copy(
                seg_hbm.at[pl.ds(row_off + mids[k] * GRP + offs[k], 16)],
                probe4.at[k], psems[k],
            ))
        out = []
        for k in range(4):
            copies[k].wait()
            pred = probe4[k, pl.ds(0, 16)][lanes[k]] >= thresholds[k]
            active = los[k] < his[k]
            out.append(jnp.where(active & jnp.logical_not(pred),
                                 mids[k] + 1, los[k]))
            out.append(jnp.where(active & pred, mids[k], his[k]))
        return tuple(out)

    init = (jnp.int32(0), jnp.int32(NGRP)) * 4
    res = lax.fori_loop(0, SEARCH_IT, sit, init)
    g1_lo, g1_hi, g2_lo, g2_hi = res[0], res[2], res[4], res[6]

    def issue(g, b):
        pltpu.async_copy(
            seg_hbm.at[pl.ds(row_off + g * GRP, GRP)], seg2.at[b], gsems[b]
        )
        pltpu.async_copy(h_hbm.at[pl.ds(g * GRP, GRP)], rows2.at[b], gsems[b])

    def wait(b):
        pltpu.make_async_copy(
            seg_hbm.at[pl.ds(0, GRP)], seg2.at[b], gsems[b]
        ).wait()
        pltpu.make_async_copy(
            h_hbm.at[pl.ds(0, GRP)], rows2.at[b], gsems[b]
        ).wait()

    def run_window(g_lo, g_hi):
        @pl.when(g_lo < g_hi)
        def _():
            issue(g_lo, 0)

        def gbody(i, carry):
            g0 = g_lo + 2 * i
            for b in range(2):
                g = g0 + b

                @pl.when(g < g_hi)
                def _():
                    @pl.when(g + 1 < g_hi)
                    def _():
                        issue(g + 1, 1 - b)

                    wait(b)
                    for j in range(GRP // 16):
                        sv = seg2[b, pl.ds(j * 16, 16)]
                        l1 = sv - base1
                        l2 = sv - base2 + HW
                        ok1 = (l1 >= 0) & (l1 < HW)
                        ok2 = (l2 >= HW) & (l2 < SEG_W)
                        idx2[b, pl.ds(j * 16, 16)] = (
                            jnp.where(ok1, l1, jnp.where(ok2, l2, SEG_W))
                            + accbase
                        )
                    pltpu.sync_copy(rows2.at[b], acc.at[idx2.at[b]], add=True)

            return carry

        lax.fori_loop(0, (g_hi - g_lo + 1) // 2, gbody, 0)

    run_window(g1_lo, g1_hi)
    run_window(jnp.maximum(g2_lo, g1_hi), g2_hi)

    pltpu.sync_copy(
        acc.at[pl.ds(accbase, HW)],
        out_hbm.at[pl.ds(base1, HW)],
    )
    pltpu.sync_copy(
        acc.at[pl.ds(accbase + HW, HW)],
        out_hbm.at[pl.ds(base2, HW)],
    )


def _head_body(p0_ref, p1_ref, w2_ref, b2_ref, w3_ref, b3_ref, w4_ref, b4_ref,
               logits_ref, probs_ref):
    s = p0_ref[...] + p1_ref[...]
    sums = s[:S, :K]
    cnt = s[:S, K:K + 1]
    mean = sums / jnp.maximum(cnt, 1.0)
    x = jnp.dot(mean, w2_ref[...], preferred_element_type=jnp.float32) + b2_ref[...]
    x = jnp.dot(x, w3_ref[...], preferred_element_type=jnp.float32) + b3_ref[...]
    x = jnp.maximum(x, 0.0)
    logits = jnp.dot(x, w4_ref[...], preferred_element_type=jnp.float32) + b4_ref[...]
    m = jnp.max(logits, axis=-1, keepdims=True)
    e = jnp.exp(logits - m)
    probs = e / jnp.sum(e, axis=-1, keepdims=True)
    logits_ref[...] = logits
    probs_ref[...] = probs


_head = pl.pallas_call(
    _head_body,
    out_shape=(
        jax.ShapeDtypeStruct((S, 2), jnp.float32),
        jax.ShapeDtypeStruct((S, 2), jnp.float32),
    ),
)


def kernel(features, segments, W1, b1, W2, b2, W3, b3, W4, b4):
    W1p = jnp.concatenate([W1, jnp.zeros((D, HP - K), jnp.float32)], axis=1)
    b1p = jnp.concatenate(
        [b1, jnp.ones((1,), jnp.float32), jnp.zeros((HP - K - 1,), jnp.float32)]
    )[None, :]
    partials = []
    for c in range(NCHUNK):
        h_c = _make_mlp1(c)(features, W1p, b1p)
        partials.append(_make_segsum(c)(h_c, segments))
    logits, probs = _head(
        partials[0], partials[1],
        W2, b2[None, :], W3, b3[None, :], W4, b4[None, :],
    )
    return (logits, probs)
